# baseline (speedup 1.0000x reference)
import math

import jax
import jax.numpy as jnp
from jax.experimental import pallas as pl
from jax.experimental.pallas import tpu as pltpu

N, E, D = 2048, 65536, 768
H, DH = 8, 96
T = 5
BQ = 512
NQ = N // BQ
BR = 512
NB = N // BR


def _layer_body(A_ref, x_ref, W_ref, aggb_ref, u1_ref, u2_ref, ub_ref,
                g_ref, bb_ref, out_ref, acc_ref):
    b = pl.program_id(0)
    t = pl.program_id(1)
    C = jnp.dot(A_ref[0], x_ref[...].astype(jnp.bfloat16),
                preferred_element_type=jnp.float32)
    contrib = jnp.dot(C, W_ref[0].T, preferred_element_type=jnp.float32)

    @pl.when(t == 0)
    def _():
        acc_ref[...] = aggb_ref[...] + contrib

    @pl.when(t > 0)
    def _():
        acc_ref[...] += contrib

    @pl.when(t == T - 1)
    def _():
        xb = x_ref[pl.ds(b * BR, BR), :]
        upd = jnp.dot(xb, u1_ref[...].T, preferred_element_type=jnp.float32)
        upd += jnp.dot(acc_ref[...], u2_ref[...].T, preferred_element_type=jnp.float32)
        upd = jnp.maximum(upd + ub_ref[...], 0.0)
        y = xb + upd
        m = jnp.mean(y, axis=-1, keepdims=True)
        v = jnp.mean((y - m) ** 2, axis=-1, keepdims=True)
        out_ref[...] = (y - m) / jnp.sqrt(v + 1e-5) * g_ref[...] + bb_ref[...]


def _gnn_layer(A3, x, W_l, aggb_l, upd_W_l, upd_b_l, g_l, b_l):
    return pl.pallas_call(
        _layer_body,
        grid=(NB, T),
        in_specs=[
            pl.BlockSpec((1, BR, N), lambda b, t: (t, b, 0)),
            pl.BlockSpec((N, D), lambda b, t: (0, 0)),
            pl.BlockSpec((1, D, D), lambda b, t: (t, 0, 0)),
            pl.BlockSpec((BR, D), lambda b, t: (b, 0)),
            pl.BlockSpec((D, D), lambda b, t: (0, 0)),
            pl.BlockSpec((D, D), lambda b, t: (0, 0)),
            pl.BlockSpec((1, D), lambda b, t: (0, 0)),
            pl.BlockSpec((1, D), lambda b, t: (0, 0)),
            pl.BlockSpec((1, D), lambda b, t: (0, 0)),
        ],
        out_specs=pl.BlockSpec((BR, D), lambda b, t: (b, 0)),
        out_shape=jax.ShapeDtypeStruct((N, D), jnp.float32),
        scratch_shapes=[pltpu.VMEM((BR, D), jnp.float32)],
    )(A3, x, W_l, aggb_l, upd_W_l[:, :D], upd_W_l[:, D:], upd_b_l[None],
      g_l[None], b_l[None])


def _msg_body(x_ref, W_ref, b_ref, y_ref):
    y_ref[0] = jnp.dot(x_ref[...], W_ref[0].T,
                       preferred_element_type=jnp.float32) + b_ref[0]


def _layer1_body(A_ref, x_ref, d_ref, u1_ref, u2_ref, ub_ref,
                 g_ref, bb_ref, out_ref, acc_ref):
    b = pl.program_id(0)
    t = pl.program_id(1)
    contrib = jnp.dot(A_ref[0], d_ref[0], preferred_element_type=jnp.float32)

    @pl.when(t == 0)
    def _():
        acc_ref[...] = contrib

    @pl.when(t > 0)
    def _():
        acc_ref[...] += contrib

    @pl.when(t == T - 1)
    def _():
        xb = x_ref[pl.ds(b * BR, BR), :]
        upd = jnp.dot(xb, u1_ref[...].T, preferred_element_type=jnp.float32)
        upd += jnp.dot(acc_ref[...], u2_ref[...].T, preferred_element_type=jnp.float32)
        upd = jnp.maximum(upd + ub_ref[...], 0.0)
        y = xb + upd
        m = jnp.mean(y, axis=-1, keepdims=True)
        v = jnp.mean((y - m) ** 2, axis=-1, keepdims=True)
        out_ref[...] = (y - m) / jnp.sqrt(v + 1e-5) * g_ref[...] + bb_ref[...]


def _gnn_layer1(A3, x, W_l, b_l, upd_W_l, upd_b_l, g_l, bb_l):
    d_tabs = pl.pallas_call(
        _msg_body,
        grid=(T,),
        in_specs=[
            pl.BlockSpec((N, D), lambda t: (0, 0)),
            pl.BlockSpec((1, D, D), lambda t: (t, 0, 0)),
            pl.BlockSpec((1, 1, D), lambda t: (t, 0, 0)),
        ],
        out_specs=pl.BlockSpec((1, N, D), lambda t: (t, 0, 0)),
        out_shape=jax.ShapeDtypeStruct((T, N, D), jnp.float32),
    )(x, W_l, b_l[:, None])
    return pl.pallas_call(
        _layer1_body,
        grid=(NB, T),
        in_specs=[
            pl.BlockSpec((1, BR, N), lambda b, t: (t, b, 0)),
            pl.BlockSpec((N, D), lambda b, t: (0, 0)),
            pl.BlockSpec((1, N, D), lambda b, t: (t, 0, 0)),
            pl.BlockSpec((D, D), lambda b, t: (0, 0)),
            pl.BlockSpec((D, D), lambda b, t: (0, 0)),
            pl.BlockSpec((1, D), lambda b, t: (0, 0)),
            pl.BlockSpec((1, D), lambda b, t: (0, 0)),
            pl.BlockSpec((1, D), lambda b, t: (0, 0)),
        ],
        out_specs=pl.BlockSpec((BR, D), lambda b, t: (b, 0)),
        out_shape=jax.ShapeDtypeStruct((N, D), jnp.float32),
        scratch_shapes=[pltpu.VMEM((BR, D), jnp.float32)],
    )(A3, x, d_tabs, upd_W_l[:, :D], upd_W_l[:, D:], upd_b_l[None],
      g_l[None], bb_l[None])


def _qkv_body(x_ref, w_ref, b_ref, o_ref):
    o_ref[0, 0] = jnp.dot(x_ref[...], w_ref[0, 0].T,
                          preferred_element_type=jnp.float32) + b_ref[0, 0]


def _attn_body(q_ref, k_ref, v_ref, o_ref, aw_ref):
    h = pl.program_id(1)
    q = q_ref[0, 0]
    k = k_ref[0, 0]
    v = v_ref[0, 0]
    s = jnp.dot(q, k.T, preferred_element_type=jnp.float32) * (1.0 / math.sqrt(DH))
    m = jnp.max(s, axis=-1, keepdims=True)
    p = jnp.exp(s - m)
    p = p * (1.0 / jnp.sum(p, axis=-1, keepdims=True))
    o_ref[0] = jnp.dot(p, v, preferred_element_type=jnp.float32)

    @pl.when(h == 0)
    def _():
        aw_ref[...] = p * (1.0 / H)

    @pl.when(h != 0)
    def _():
        aw_ref[...] += p * (1.0 / H)


def _out_body(o_ref, w_ref, b_ref, gr_ref, pool_ref):
    h = pl.program_id(0)
    contrib = jnp.dot(o_ref[0], w_ref[0], preferred_element_type=jnp.float32)

    @pl.when(h == 0)
    def _():
        gr_ref[...] = contrib + b_ref[...]

    @pl.when(h > 0)
    def _():
        gr_ref[...] += contrib

    @pl.when(h == H - 1)
    def _():
        pool_ref[...] = jnp.sum(gr_ref[...], axis=0, keepdims=True) * (1.0 / N)


def _attention(x, in_proj_W, in_proj_b, out_W, out_b):
    qkv = pl.pallas_call(
        _qkv_body,
        grid=(3, H),
        in_specs=[
            pl.BlockSpec((N, D), lambda p, h: (0, 0)),
            pl.BlockSpec((1, 1, DH, D), lambda p, h: (p, h, 0, 0)),
            pl.BlockSpec((1, 1, 1, DH), lambda p, h: (p, h, 0, 0)),
        ],
        out_specs=pl.BlockSpec((1, 1, N, DH), lambda p, h: (p, h, 0, 0)),
        out_shape=jax.ShapeDtypeStruct((3, H, N, DH), jnp.float32),
    )(x, in_proj_W.reshape(3, H, DH, D), in_proj_b.reshape(3, H, 1, DH))
    o, aw = pl.pallas_call(
        _attn_body,
        grid=(NQ, H),
        in_specs=[
            pl.BlockSpec((1, 1, BQ, DH), lambda i, h: (0, h, i, 0)),
            pl.BlockSpec((1, 1, N, DH), lambda i, h: (1, h, 0, 0)),
            pl.BlockSpec((1, 1, N, DH), lambda i, h: (2, h, 0, 0)),
        ],
        out_specs=[
            pl.BlockSpec((1, BQ, DH), lambda i, h: (h, i, 0)),
            pl.BlockSpec((BQ, N), lambda i, h: (i, 0)),
        ],
        out_shape=[
            jax.ShapeDtypeStruct((H, N, DH), jnp.float32),
            jax.ShapeDtypeStruct((N, N), jnp.float32),
        ],
    )(qkv, qkv, qkv)
    graph_repr, pooled = pl.pallas_call(
        _out_body,
        grid=(H,),
        in_specs=[
            pl.BlockSpec((1, N, DH), lambda h: (h, 0, 0)),
            pl.BlockSpec((1, DH, D), lambda h: (h, 0, 0)),
            pl.BlockSpec((1, D), lambda h: (0, 0)),
        ],
        out_specs=[
            pl.BlockSpec((N, D), lambda h: (0, 0)),
            pl.BlockSpec((1, D), lambda h: (0, 0)),
        ],
        out_shape=[
            jax.ShapeDtypeStruct((N, D), jnp.float32),
            jax.ShapeDtypeStruct((1, D), jnp.float32),
        ],
    )(o, out_W.T.reshape(H, DH, D), out_b[None])
    return graph_repr, pooled[0], aw[None]


def kernel(node_features, edge_indices, edge_types, node_types, statute_emb, case_emb,
           article_emb, msg_W, msg_b, upd_W, upd_b, ln_g, ln_b, in_proj_W, in_proj_b,
           out_W, out_b, a1_W, a1_b, a2_W, a2_b, a3_W, a3_b):
    e0 = jnp.concatenate([statute_emb, statute_emb, statute_emb[:N - 2000]])
    e1 = case_emb[:N]
    e2 = jnp.concatenate([article_emb, article_emb[:N - 2000]])
    nt = node_types[:, None]
    node_emb = jnp.where(nt == 0, e0, jnp.where(nt == 1, e1, e2))
    x = node_features + node_emb

    src = edge_indices[0]
    dst = edge_indices[1]
    flat = (edge_types * N + dst) * N + src
    A3b = (jnp.zeros((T * N * N,), jnp.bfloat16).at[flat].add(jnp.bfloat16(1))
           .reshape(T, N, N))
    deg = A3b.sum(axis=2, dtype=jnp.float32).T

    for l in range(3):
        aggb_l = deg @ msg_b[l]
        x = _gnn_layer(A3b, x, msg_W[l], aggb_l, upd_W[l], upd_b[l],
                       ln_g[l], ln_b[l])

    graph_repr, pooled, attn_weights = _attention(x, in_proj_W, in_proj_b,
                                                  out_W, out_b)
    h = jax.nn.relu(pooled @ a1_W.T + a1_b)
    h = jax.nn.relu(h @ a2_W.T + a2_b)
    score = jax.nn.sigmoid(h @ a3_W.T + a3_b)
    return x, graph_repr, score, attn_weights


# speedup vs baseline: 1.3945x; 1.3945x over previous
import math

import jax
import jax.numpy as jnp
from jax.experimental import pallas as pl
from jax.experimental.pallas import tpu as pltpu

N, E, D = 2048, 65536, 768
H, DH = 8, 96
T = 5
BQ = 1024
NQ = N // BQ
BR = 512
NB = N // BR


def _layer_body(A_ref, x_ref, W_ref, aggb_ref, u1_ref, u2_ref, ub_ref,
                g_ref, bb_ref, out_ref, acc_ref):
    b = pl.program_id(0)
    t = pl.program_id(1)
    C = jnp.dot(A_ref[0], x_ref[...], preferred_element_type=jnp.float32)
    contrib = jnp.dot(C, W_ref[0].T, preferred_element_type=jnp.float32)

    @pl.when(t == 0)
    def _():
        acc_ref[...] = aggb_ref[...] + contrib

    @pl.when(t > 0)
    def _():
        acc_ref[...] += contrib

    @pl.when(t == T - 1)
    def _():
        xb = x_ref[pl.ds(b * BR, BR), :]
        upd = jnp.dot(xb, u1_ref[...].T, preferred_element_type=jnp.float32)
        upd += jnp.dot(acc_ref[...], u2_ref[...].T, preferred_element_type=jnp.float32)
        upd = jnp.maximum(upd + ub_ref[...], 0.0)
        y = xb + upd
        m = jnp.mean(y, axis=-1, keepdims=True)
        v = jnp.mean((y - m) ** 2, axis=-1, keepdims=True)
        out_ref[...] = (y - m) / jnp.sqrt(v + 1e-5) * g_ref[...] + bb_ref[...]


def _gnn_layer(A3, x, W_l, aggb_l, upd_W_l, upd_b_l, g_l, b_l):
    return pl.pallas_call(
        _layer_body,
        grid=(NB, T),
        in_specs=[
            pl.BlockSpec((1, BR, N), lambda b, t: (t, b, 0)),
            pl.BlockSpec((N, D), lambda b, t: (0, 0)),
            pl.BlockSpec((1, D, D), lambda b, t: (t, 0, 0)),
            pl.BlockSpec((BR, D), lambda b, t: (b, 0)),
            pl.BlockSpec((D, D), lambda b, t: (0, 0)),
            pl.BlockSpec((D, D), lambda b, t: (0, 0)),
            pl.BlockSpec((1, D), lambda b, t: (0, 0)),
            pl.BlockSpec((1, D), lambda b, t: (0, 0)),
            pl.BlockSpec((1, D), lambda b, t: (0, 0)),
        ],
        out_specs=pl.BlockSpec((BR, D), lambda b, t: (b, 0)),
        out_shape=jax.ShapeDtypeStruct((N, D), jnp.float32),
        scratch_shapes=[pltpu.VMEM((BR, D), jnp.float32)],
    )(A3, x, W_l, aggb_l, upd_W_l[:, :D], upd_W_l[:, D:], upd_b_l[None],
      g_l[None], b_l[None])


def _qkv_body(x_ref, w_ref, b_ref, o_ref):
    o_ref[0, 0] = jnp.dot(x_ref[...], w_ref[0, 0].T,
                          preferred_element_type=jnp.float32) + b_ref[0, 0]


def _attn_body(q_ref, k_ref, v_ref, o_ref, aw_ref):
    h = pl.program_id(1)
    q = q_ref[0, 0]
    k = k_ref[0, 0]
    v = v_ref[0, 0]
    s = jnp.dot(q, k.T, preferred_element_type=jnp.float32) * (1.0 / math.sqrt(DH))
    m = jnp.max(s, axis=-1, keepdims=True)
    p = jnp.exp(s - m)
    p = p * (1.0 / jnp.sum(p, axis=-1, keepdims=True))
    o_ref[0] = jnp.dot(p, v, preferred_element_type=jnp.float32)

    @pl.when(h == 0)
    def _():
        aw_ref[...] = p * (1.0 / H)

    @pl.when(h != 0)
    def _():
        aw_ref[...] += p * (1.0 / H)


def _out_body(o_ref, w_ref, b_ref, gr_ref, pool_ref):
    h = pl.program_id(0)
    contrib = jnp.dot(o_ref[0], w_ref[0], preferred_element_type=jnp.float32)

    @pl.when(h == 0)
    def _():
        gr_ref[...] = contrib + b_ref[...]

    @pl.when(h > 0)
    def _():
        gr_ref[...] += contrib

    @pl.when(h == H - 1)
    def _():
        pool_ref[...] = jnp.sum(gr_ref[...], axis=0, keepdims=True) * (1.0 / N)


def _attention(x, in_proj_W, in_proj_b, out_W, out_b):
    qkv = pl.pallas_call(
        _qkv_body,
        grid=(3, H),
        in_specs=[
            pl.BlockSpec((N, D), lambda p, h: (0, 0)),
            pl.BlockSpec((1, 1, DH, D), lambda p, h: (p, h, 0, 0)),
            pl.BlockSpec((1, 1, 1, DH), lambda p, h: (p, h, 0, 0)),
        ],
        out_specs=pl.BlockSpec((1, 1, N, DH), lambda p, h: (p, h, 0, 0)),
        out_shape=jax.ShapeDtypeStruct((3, H, N, DH), jnp.float32),
    )(x, in_proj_W.reshape(3, H, DH, D), in_proj_b.reshape(3, H, 1, DH))
    o, aw = pl.pallas_call(
        _attn_body,
        grid=(NQ, H),
        in_specs=[
            pl.BlockSpec((1, 1, BQ, DH), lambda i, h: (0, h, i, 0)),
            pl.BlockSpec((1, 1, N, DH), lambda i, h: (1, h, 0, 0)),
            pl.BlockSpec((1, 1, N, DH), lambda i, h: (2, h, 0, 0)),
        ],
        out_specs=[
            pl.BlockSpec((1, BQ, DH), lambda i, h: (h, i, 0)),
            pl.BlockSpec((BQ, N), lambda i, h: (i, 0)),
        ],
        out_shape=[
            jax.ShapeDtypeStruct((H, N, DH), jnp.float32),
            jax.ShapeDtypeStruct((N, N), jnp.float32),
        ],
    )(qkv, qkv, qkv)
    graph_repr, pooled = pl.pallas_call(
        _out_body,
        grid=(H,),
        in_specs=[
            pl.BlockSpec((1, N, DH), lambda h: (h, 0, 0)),
            pl.BlockSpec((1, DH, D), lambda h: (h, 0, 0)),
            pl.BlockSpec((1, D), lambda h: (0, 0)),
        ],
        out_specs=[
            pl.BlockSpec((N, D), lambda h: (0, 0)),
            pl.BlockSpec((1, D), lambda h: (0, 0)),
        ],
        out_shape=[
            jax.ShapeDtypeStruct((N, D), jnp.float32),
            jax.ShapeDtypeStruct((1, D), jnp.float32),
        ],
    )(o, out_W.T.reshape(H, DH, D), out_b[None])
    return graph_repr, pooled[0], aw[None]


def kernel(node_features, edge_indices, edge_types, node_types, statute_emb, case_emb,
           article_emb, msg_W, msg_b, upd_W, upd_b, ln_g, ln_b, in_proj_W, in_proj_b,
           out_W, out_b, a1_W, a1_b, a2_W, a2_b, a3_W, a3_b):
    e0 = jnp.concatenate([statute_emb, statute_emb, statute_emb[:N - 2000]])
    e1 = case_emb[:N]
    e2 = jnp.concatenate([article_emb, article_emb[:N - 2000]])
    nt = node_types[:, None]
    node_emb = jnp.where(nt == 0, e0, jnp.where(nt == 1, e1, e2))
    x = node_features + node_emb

    src = edge_indices[0]
    dst = edge_indices[1]
    flat = (edge_types * N + dst) * N + src
    A3 = (jnp.zeros((T * N * N,), jnp.float32).at[flat].add(1.0)
          .reshape(T, N, N))
    deg = A3.sum(axis=2).T

    for l in range(3):
        aggb_l = deg @ msg_b[l]
        x = _gnn_layer(A3, x, msg_W[l], aggb_l, upd_W[l], upd_b[l],
                       ln_g[l], ln_b[l])

    graph_repr, pooled, attn_weights = _attention(x, in_proj_W, in_proj_b,
                                                  out_W, out_b)
    h = jax.nn.relu(pooled @ a1_W.T + a1_b)
    h = jax.nn.relu(h @ a2_W.T + a2_b)
    score = jax.nn.sigmoid(h @ a3_W.T + a3_b)
    return x, graph_repr, score, attn_weights
